# baseline (speedup 1.0000x reference)
import functools

import jax
import jax.numpy as jnp
from jax import lax
from jax.experimental import pallas as pl
from jax.experimental.pallas import tpu as pltpu
from jax.experimental.pallas import tpu_sc as plsc

D = 64
B = 4096
L = 200
HID = 300
NCLS = 100
V = 1_000_000

NC = 2
NS = 16
NW = NC * NS

CW = 256
NCH_T = V // CW
TAIL0 = NCH_T * CW
TAILW = V - TAIL0
CPT = (NCH_T + NW - 1) // NW


def _tr_body(tt_hbm, out_hbm, ibuf, obuf, tbuf, isems, osems):
    wid = lax.axis_index("s") * NC + lax.axis_index("c")
    iot = lax.iota(jnp.int32, 16)
    qbase = lax.shift_right_logical(iot, 1)
    zbase = lax.shift_left(lax.bitwise_and(iot, 1), 6)
    qvs = [qbase + g * 8 for g in range(CW // 16)]

    def chunk_of(k):
        return lax.min(wid + k * NW, NCH_T - 1)

    def in_cp(k, slot):
        c = chunk_of(k)
        return pltpu.make_async_copy(
            tt_hbm.at[:, pl.ds(c * CW, CW)], ibuf.at[slot], isems.at[slot])

    def out_cp(k, slot):
        c = chunk_of(k)
        return pltpu.make_async_copy(
            obuf.at[slot], out_hbm.at[pl.ds(c * (CW // 2), CW // 2)],
            osems.at[slot])

    in_cp(0, 0).start()
    in_cp(1, 1).start()

    def step(k, carry):
        slot = lax.rem(k, 2)
        in_cp(k, slot).wait()

        @pl.when(k >= 2)
        def _():
            out_cp(k - 2, slot).wait()

        def drow(d, c2):
            zv = zbase + d
            for g in range(CW // 16):
                v = ibuf[slot, d, pl.ds(g * 16, 16)]
                plsc.store_scatter(
                    obuf, [jnp.full((16,), slot, jnp.int32), qvs[g], zv], v)
            return c2

        lax.fori_loop(0, D, drow, 0)

        @pl.when(k + 2 < CPT)
        def _():
            in_cp(k + 2, slot).start()

        out_cp(k, slot).start()
        return carry

    lax.fori_loop(0, CPT, step, 0)
    out_cp(CPT - 2, lax.rem(CPT, 2)).wait()
    out_cp(CPT - 1, lax.rem(CPT + 1, 2)).wait()

    @pl.when(wid == 0)
    def _():
        pltpu.sync_copy(tt_hbm.at[:, pl.ds(TAIL0, TAILW)], tbuf)

        def drow_t(d, c2):
            zv = zbase + d
            for g in range(TAILW // 16):
                v = tbuf[d, pl.ds(g * 16, 16)]
                plsc.store_scatter(
                    obuf, [jnp.zeros((16,), jnp.int32), qvs[g], zv], v)
            return c2

        lax.fori_loop(0, D, drow_t, 0)
        pltpu.sync_copy(obuf.at[0, pl.ds(0, TAILW // 2)],
                        out_hbm.at[pl.ds(TAIL0 // 2, TAILW // 2)])


_transpose = functools.partial(
    pl.kernel,
    out_type=jax.ShapeDtypeStruct((V // 2, 2 * D), jnp.float32),
    mesh=plsc.VectorSubcoreMesh(core_axis_name="c", subcore_axis_name="s"),
    compiler_params=pltpu.CompilerParams(needs_layout_passes=False),
    scratch_types=[
        pltpu.VMEM((2, D, CW), jnp.float32),
        pltpu.VMEM((2, CW // 2, 2 * D), jnp.float32),
        pltpu.VMEM((D, TAILW), jnp.float32),
        pltpu.SemaphoreType.DMA((2,)),
        pltpu.SemaphoreType.DMA((2,)),
    ],
)(_tr_body)


ROWS_PER = B // NW
TOK_PER = ROWS_PER * L
C0, C1 = 104, 96
NBUF = 7


def _pool_body(x_hbm, table_hbm, out_hbm, idx_v, rows_v, acc_v, sems):
    wid = lax.axis_index("s") * NC + lax.axis_index("c")
    pltpu.sync_copy(x_hbm.at[wid], idx_v)

    def gather(r, h, slot):
        base = r * L + h * C0
        ln = C1 if h else C0
        return pltpu.make_async_copy(
            table_hbm.at[idx_v.at[pl.ds(base, ln)]],
            rows_v.at[slot, pl.ds(0, ln)], sems.at[slot])

    for j in range(NBUF - 1):
        gather(j // 2, j % 2, j % NBUF).start()

    def row_body(r, carry):
        accs = (jnp.zeros((16,), jnp.float32),) * 4
        for h in range(2):
            j = r * 2 + h
            slot = lax.rem(j, NBUF)
            gather(r, h, slot).wait()

            @pl.when(j + NBUF - 1 < 2 * ROWS_PER)
            def _():
                jn = j + NBUF - 1
                gather(jn // 2, h, lax.rem(jn, NBUF)).start()

            ln = C1 if h else C0

            def tacc(t, a):
                t0 = t * 8
                out = list(a)
                for u in range(8):
                    for k in range(4):
                        out[k] = out[k] + rows_v[slot, t0 + u,
                                                 pl.ds(k * 16, 16)]
                return tuple(out)

            accs = lax.fori_loop(0, ln // 8, tacc, accs)
        for k in range(4):
            acc_v[r, pl.ds(k * 16, 16)] = accs[k]
        return carry

    lax.fori_loop(0, ROWS_PER, row_body, 0)
    pltpu.sync_copy(acc_v, out_hbm.at[wid])


_pool = functools.partial(
    pl.kernel,
    out_type=jax.ShapeDtypeStruct((NW, ROWS_PER, D), jnp.float32),
    mesh=plsc.VectorSubcoreMesh(core_axis_name="c", subcore_axis_name="s"),
    compiler_params=pltpu.CompilerParams(use_tc_tiling_on_sc=False),
    scratch_types=[
        pltpu.VMEM((TOK_PER,), jnp.int32),
        pltpu.VMEM((NBUF, C0, D), jnp.float32),
        pltpu.VMEM((ROWS_PER, D), jnp.float32),
        pltpu.SemaphoreType.DMA((NBUF,)),
    ],
)(_pool_body)


def _mlp_body(p_ref, w1_ref, b1_ref, w2_ref, b2_ref, o_ref):
    h = jnp.dot(p_ref[...] * (1.0 / L), w1_ref[...],
                preferred_element_type=jnp.float32) + b1_ref[...]
    h = jnp.maximum(h, 0.0)
    o_ref[...] = jnp.dot(h, w2_ref[...],
                         preferred_element_type=jnp.float32) + b2_ref[...]


def _mlp(pooled, W1, b1, W2, b2):
    return pl.pallas_call(
        _mlp_body,
        out_shape=jax.ShapeDtypeStruct((B, NCLS), jnp.float32),
    )(pooled, W1, b1.reshape(1, HID), W2, b2.reshape(1, NCLS))


def kernel(x, table, W1, b1, W2, b2):
    x2 = x.astype(jnp.int32).reshape(NW, TOK_PER)
    twide = _transpose(table.T)
    t64 = twide.reshape(V, D)
    pooled = _pool(x2, t64)
    return _mlp(pooled.reshape(B, D), W1, b1, W2, b2)


# speedup vs baseline: 1.3608x; 1.3608x over previous
import functools

import jax
import jax.numpy as jnp
from jax import lax
from jax.experimental import pallas as pl
from jax.experimental.pallas import tpu as pltpu
from jax.experimental.pallas import tpu_sc as plsc

D = 64
B = 4096
L = 200
HID = 300
NCLS = 100
V = 1_000_000

NC = 2
NS = 16
NW = NC * NS

CW = 256
NCH_T = V // CW
TAIL0 = NCH_T * CW
TAILW = V - TAIL0
CPT = (NCH_T + NW - 1) // NW


def _tr_body(tt_hbm, tailw_hbm, out_hbm, ibuf, obuf, scr, isems, osems):
    wid = lax.axis_index("s") * NC + lax.axis_index("c")
    iot = lax.iota(jnp.int32, 16)
    sidx = [iot * 17 + r for r in range(16)]

    def chunk_of(k):
        return lax.min(wid + k * NW, NCH_T - 1)

    def in_cp(k, slot):
        c = chunk_of(k)
        return pltpu.make_async_copy(
            tt_hbm.at[:, pl.ds(c * CW, CW)], ibuf.at[slot], isems.at[slot])

    def out_cp(k, slot):
        c = chunk_of(k)
        return pltpu.make_async_copy(
            obuf.at[slot], out_hbm.at[pl.ds(c * (CW // 2), CW // 2)],
            osems.at[slot])

    def block_xpose(slot, jb, nbj):
        def bj_body(bj, c2):
            j0 = jb + bj * 16
            for bd in range(4):
                d0 = bd * 16
                for r in range(16):
                    v = ibuf[slot, d0 + r, pl.ds(j0, 16)]
                    plsc.store_scatter(scr, [sidx[r]], v)
                for jl in range(16):
                    tv = scr[pl.ds(jl * 17, 16)]
                    ql = lax.shift_right_logical(j0 + jl, 1)
                    obuf[slot, ql, pl.ds((jl & 1) * 64 + d0, 16)] = tv
            return c2
        lax.fori_loop(0, nbj, bj_body, 0)

    in_cp(0, 0).start()
    in_cp(1, 1).start()

    def step(k, carry):
        slot = lax.rem(k, 2)
        in_cp(k, slot).wait()

        @pl.when(k >= 2)
        def _():
            out_cp(k - 2, slot).wait()

        block_xpose(slot, 0, CW // 16)

        @pl.when(k + 2 < CPT)
        def _():
            in_cp(k + 2, slot).start()

        out_cp(k, slot).start()
        return carry

    lax.fori_loop(0, CPT, step, 0)
    out_cp(CPT - 2, lax.rem(CPT, 2)).wait()
    out_cp(CPT - 1, lax.rem(CPT + 1, 2)).wait()

    @pl.when(wid == 0)
    def _():
        pltpu.sync_copy(tailw_hbm, out_hbm.at[pl.ds(TAIL0 // 2, TAILW // 2)])


_transpose = functools.partial(
    pl.kernel,
    out_type=jax.ShapeDtypeStruct((V // 2, 2 * D), jnp.float32),
    mesh=plsc.VectorSubcoreMesh(core_axis_name="c", subcore_axis_name="s"),
    compiler_params=pltpu.CompilerParams(needs_layout_passes=False),
    scratch_types=[
        pltpu.VMEM((2, D, CW), jnp.float32),
        pltpu.VMEM((2, CW // 2, 2 * D), jnp.float32),
        pltpu.VMEM((272,), jnp.float32),
        pltpu.SemaphoreType.DMA((2,)),
        pltpu.SemaphoreType.DMA((2,)),
    ],
)(_tr_body)


ROWS_PER = B // NW
TOK_PER = ROWS_PER * L
C0, C1 = 104, 96
NBUF = 7


def _pool_body(x_hbm, table_hbm, out_hbm, idx_v, rows_v, acc_v, sems):
    wid = lax.axis_index("s") * NC + lax.axis_index("c")
    pltpu.sync_copy(x_hbm.at[wid], idx_v)

    def gather(r, h, slot):
        base = r * L + h * C0
        ln = C1 if h else C0
        return pltpu.make_async_copy(
            table_hbm.at[idx_v.at[pl.ds(base, ln)]],
            rows_v.at[slot, pl.ds(0, ln)], sems.at[slot])

    for j in range(NBUF - 1):
        gather(j // 2, j % 2, j % NBUF).start()

    def row_body(r, carry):
        accs = (jnp.zeros((16,), jnp.float32),) * 4
        for h in range(2):
            j = r * 2 + h
            slot = lax.rem(j, NBUF)
            gather(r, h, slot).wait()

            @pl.when(j + NBUF - 1 < 2 * ROWS_PER)
            def _():
                jn = j + NBUF - 1
                gather(jn // 2, h, lax.rem(jn, NBUF)).start()

            ln = C1 if h else C0

            def tacc(t, a):
                t0 = t * 8
                out = list(a)
                for u in range(8):
                    for k in range(4):
                        out[k] = out[k] + rows_v[slot, t0 + u,
                                                 pl.ds(k * 16, 16)]
                return tuple(out)

            accs = lax.fori_loop(0, ln // 8, tacc, accs)
        for k in range(4):
            acc_v[r, pl.ds(k * 16, 16)] = accs[k]
        return carry

    lax.fori_loop(0, ROWS_PER, row_body, 0)
    pltpu.sync_copy(acc_v, out_hbm.at[wid])


_pool = functools.partial(
    pl.kernel,
    out_type=jax.ShapeDtypeStruct((NW, ROWS_PER, D), jnp.float32),
    mesh=plsc.VectorSubcoreMesh(core_axis_name="c", subcore_axis_name="s"),
    compiler_params=pltpu.CompilerParams(use_tc_tiling_on_sc=False),
    scratch_types=[
        pltpu.VMEM((TOK_PER,), jnp.int32),
        pltpu.VMEM((NBUF, C0, D), jnp.float32),
        pltpu.VMEM((ROWS_PER, D), jnp.float32),
        pltpu.SemaphoreType.DMA((NBUF,)),
    ],
)(_pool_body)


def _mlp_body(p_ref, w1_ref, b1_ref, w2_ref, b2_ref, o_ref):
    h = jnp.dot(p_ref[...] * (1.0 / L), w1_ref[...],
                preferred_element_type=jnp.float32) + b1_ref[...]
    h = jnp.maximum(h, 0.0)
    o_ref[...] = jnp.dot(h, w2_ref[...],
                         preferred_element_type=jnp.float32) + b2_ref[...]


def _mlp(pooled, W1, b1, W2, b2):
    return pl.pallas_call(
        _mlp_body,
        out_shape=jax.ShapeDtypeStruct((B, NCLS), jnp.float32),
    )(pooled, W1, b1.reshape(1, HID), W2, b2.reshape(1, NCLS))


def kernel(x, table, W1, b1, W2, b2):
    x2 = x.astype(jnp.int32).reshape(NW, TOK_PER)
    tailw = table[TAIL0:, :].reshape(TAILW // 2, 2 * D)
    twide = _transpose(table.T, tailw)
    t64 = twide.reshape(V, D)
    pooled = _pool(x2, t64)
    return _mlp(pooled.reshape(B, D), W1, b1, W2, b2)
